# Initial kernel scaffold; baseline (speedup 1.0000x reference)
import jax
import jax.numpy as jnp
from jax.experimental import pallas as pl


def kernel(x, edge_index, W_l1, b_l1, W_r1, gamma1, beta1, W_l2, b_l2, W_r2, gamma2, beta2, W_l3, b_l3, W_r3, gamma3, beta3):
    raise NotImplementedError("write your pallas kernel here")


# speedup vs baseline: 7.9870x; 7.9870x over previous
import functools

import jax
import jax.numpy as jnp
from jax import lax
from jax.experimental import pallas as pl
from jax.experimental.pallas import tpu as pltpu
from jax.experimental.pallas import tpu_sc as plsc

NC = 2
NS = 16
NW = NC * NS
CH = 128
LANES = 16


def _ceil_div(a, b):
    return -(-a // b)


def _chunk_starts(nchunk, nworkers):
    starts = [((w * nchunk) // nworkers) // 8 * 8 for w in range(nworkers)]
    ends = starts[1:] + [nchunk]
    counts = [e - s for s, e in zip(starts, ends)]
    maxc = _ceil_div(max(counts), 8) * 8
    need_rows = max(s + maxc for s in starts)
    return need_rows, maxc


def _make_agg_kernel(N, E, D):
    nchunk = E // CH
    assert nchunk * CH == E
    npad = _ceil_div(N, CH * NS) * (CH * NS)
    rpt = npad // NS
    _, maxc = _chunk_starts(nchunk, NW)
    zr = rpt
    ncp = 1
    while zr > CH:
        for d in (5, 4, 3, 2):
            if zr % d == 0:
                zr //= d
                ncp *= d
                break
        else:
            raise ValueError(rpt)

    mesh = plsc.VectorSubcoreMesh(core_axis_name="c", subcore_axis_name="s")

    @functools.partial(
        pl.kernel,
        out_type=jax.ShapeDtypeStruct((NC, npad, D), jnp.float32),
        mesh=mesh,
        scratch_types=[
            pltpu.VMEM((maxc, CH), jnp.int32),
            pltpu.VMEM((maxc, CH), jnp.int32),
            pltpu.VMEM((CH, D), jnp.float32),
            pltpu.VMEM_SHARED((npad, D), jnp.float32),
            pltpu.SemaphoreType.DMA,
        ],
    )
    def agg_kernel(tbl_hbm, src_hbm, dst_hbm, out_hbm, src_v, dst_v, rows_v,
                   acc_sh, sem):
        c = lax.axis_index("c")
        s = lax.axis_index("s")
        w = c * NS + s

        zeros16 = jnp.zeros((LANES,), jnp.float32)

        def zero_body(i, _):
            for j in range(D // LANES):
                rows_v[i, pl.ds(j * LANES, LANES)] = zeros16
            return _

        lax.fori_loop(0, zr, zero_body, None)
        for k in range(ncp):
            pltpu.sync_copy(rows_v.at[pl.ds(0, zr)],
                            acc_sh.at[pl.ds(s * rpt + k * zr, zr)])
        plsc.subcore_barrier()

        start = ((w * nchunk) // NW) // 8 * 8
        nxt = (((w + 1) * nchunk) // NW) // 8 * 8
        count = jnp.where(w == NW - 1, nchunk, nxt) - start
        pltpu.sync_copy(src_hbm.at[pl.ds(start, maxc)], src_v)
        pltpu.sync_copy(dst_hbm.at[pl.ds(start, maxc)], dst_v)

        def chunk_body(i, _):
            pltpu.async_copy(tbl_hbm.at[src_v.at[i]], rows_v, sem).wait()
            pltpu.sync_copy(rows_v, acc_sh.at[dst_v.at[i]], add=True)
            return _

        lax.fori_loop(0, count, chunk_body, None)
        plsc.subcore_barrier()
        pltpu.sync_copy(acc_sh.at[pl.ds(s * rpt, rpt)],
                        out_hbm.at[c, pl.ds(s * rpt, rpt)])

    return agg_kernel


BN_ROWS = 1000


def _dot(a, b):
    return jnp.dot(a, b, preferred_element_type=jnp.float32)


def _stats_tail(o, out_ref, s1_ref, s2_ref):
    out_ref[...] = o

    @pl.when(pl.program_id(0) == 0)
    def _():
        s1_ref[...] = jnp.zeros_like(s1_ref)
        s2_ref[...] = jnp.zeros_like(s2_ref)

    s1_ref[...] += jnp.sum(o, axis=0, keepdims=True)
    s2_ref[...] += jnp.sum(o * o, axis=0, keepdims=True)


def _layer1_a(N, D_in, D_out):
    nb = N // BN_ROWS

    def body(agg_ref, cnt_ref, x_ref, wl_ref, wr_ref, bl_ref,
             out_ref, s1_ref, s2_ref, inv_ref):
        cnt = cnt_ref[0, :, :1] + cnt_ref[1, :, :1]
        inv = 1.0 / jnp.maximum(cnt, 1.0)
        inv_ref[...] = inv
        mean = (agg_ref[0] + agg_ref[1]) * inv
        o = _dot(mean, wl_ref[...]) + _dot(x_ref[...], wr_ref[...]) + bl_ref[...]
        _stats_tail(o, out_ref, s1_ref, s2_ref)

    return pl.pallas_call(
        body,
        grid=(nb,),
        in_specs=[
            pl.BlockSpec((NC, BN_ROWS, D_in), lambda i: (0, i, 0)),
            pl.BlockSpec((NC, BN_ROWS, D_in), lambda i: (0, i, 0)),
            pl.BlockSpec((BN_ROWS, D_in), lambda i: (i, 0)),
            pl.BlockSpec((D_in, D_out), lambda i: (0, 0)),
            pl.BlockSpec((D_in, D_out), lambda i: (0, 0)),
            pl.BlockSpec((1, D_out), lambda i: (0, 0)),
        ],
        out_specs=[
            pl.BlockSpec((BN_ROWS, D_out), lambda i: (i, 0)),
            pl.BlockSpec((1, D_out), lambda i: (0, 0)),
            pl.BlockSpec((1, D_out), lambda i: (0, 0)),
            pl.BlockSpec((BN_ROWS, 1), lambda i: (i, 0)),
        ],
        out_shape=[
            jax.ShapeDtypeStruct((N, D_out), jnp.float32),
            jax.ShapeDtypeStruct((1, D_out), jnp.float32),
            jax.ShapeDtypeStruct((1, D_out), jnp.float32),
            jax.ShapeDtypeStruct((N, 1), jnp.float32),
        ],
    )


def _layer_a_nomm(N, D, nparts):
    nb = N // BN_ROWS

    def body(*refs):
        agg_refs = refs[:nparts]
        inv_ref, r_ref, bl_ref, out_ref, s1_ref, s2_ref = refs[nparts:]
        inv = inv_ref[...]
        parts = [(a[0] + a[1]) * inv for a in agg_refs]
        mean = parts[0] if nparts == 1 else jnp.concatenate(parts, axis=1)
        o = mean + r_ref[...] + bl_ref[...]
        _stats_tail(o, out_ref, s1_ref, s2_ref)

    dsub = D // nparts
    return pl.pallas_call(
        body,
        grid=(nb,),
        in_specs=[pl.BlockSpec((NC, BN_ROWS, dsub), lambda i: (0, i, 0))
                  for _ in range(nparts)] + [
            pl.BlockSpec((BN_ROWS, 1), lambda i: (i, 0)),
            pl.BlockSpec((BN_ROWS, D), lambda i: (i, 0)),
            pl.BlockSpec((1, D), lambda i: (0, 0)),
        ],
        out_specs=[
            pl.BlockSpec((BN_ROWS, D), lambda i: (i, 0)),
            pl.BlockSpec((1, D), lambda i: (0, 0)),
            pl.BlockSpec((1, D), lambda i: (0, 0)),
        ],
        out_shape=[
            jax.ShapeDtypeStruct((N, D), jnp.float32),
            jax.ShapeDtypeStruct((1, D), jnp.float32),
            jax.ShapeDtypeStruct((1, D), jnp.float32),
        ],
    )


def _bn_relu(pre_ref, s1_ref, s2_ref, g_ref, b_ref, n_nodes):
    mu = s1_ref[...] * (1.0 / n_nodes)
    var = s2_ref[...] * (1.0 / n_nodes) - mu * mu
    scale = g_ref[...] * lax.rsqrt(var + 1e-5)
    shift = b_ref[...] - mu * scale
    return jnp.maximum(pre_ref[...] * scale + shift, 0.0)


def _layer_b_next(N, D, D_next, split_p):
    nb = N // BN_ROWS
    nsplit = 2 if split_p else 1
    dsub = D_next // nsplit

    def body(pre_ref, s1_ref, s2_ref, g_ref, b_ref, wl_ref, wr_ref, *out_refs):
        h = _bn_relu(pre_ref, s1_ref, s2_ref, g_ref, b_ref, N)
        p = _dot(h, wl_ref[...])
        r = _dot(h, wr_ref[...])
        for k in range(nsplit):
            out_refs[k][...] = p[:, k * dsub:(k + 1) * dsub]
        out_refs[nsplit][...] = r

    return pl.pallas_call(
        body,
        grid=(nb,),
        in_specs=[
            pl.BlockSpec((BN_ROWS, D), lambda i: (i, 0)),
            pl.BlockSpec((1, D), lambda i: (0, 0)),
            pl.BlockSpec((1, D), lambda i: (0, 0)),
            pl.BlockSpec((1, D), lambda i: (0, 0)),
            pl.BlockSpec((1, D), lambda i: (0, 0)),
            pl.BlockSpec((D, D_next), lambda i: (0, 0)),
            pl.BlockSpec((D, D_next), lambda i: (0, 0)),
        ],
        out_specs=[pl.BlockSpec((BN_ROWS, dsub), lambda i: (i, 0))
                   for _ in range(nsplit)] +
                  [pl.BlockSpec((BN_ROWS, D_next), lambda i: (i, 0))],
        out_shape=[jax.ShapeDtypeStruct((N, dsub), jnp.float32)
                   for _ in range(nsplit)] +
                  [jax.ShapeDtypeStruct((N, D_next), jnp.float32)],
    )


def _layer_b_final(N, D):
    nb = N // BN_ROWS

    def body(pre_ref, s1_ref, s2_ref, g_ref, b_ref, out_ref):
        out_ref[...] = _bn_relu(pre_ref, s1_ref, s2_ref, g_ref, b_ref, N)

    return pl.pallas_call(
        body,
        grid=(nb,),
        in_specs=[
            pl.BlockSpec((BN_ROWS, D), lambda i: (i, 0)),
            pl.BlockSpec((1, D), lambda i: (0, 0)),
            pl.BlockSpec((1, D), lambda i: (0, 0)),
            pl.BlockSpec((1, D), lambda i: (0, 0)),
            pl.BlockSpec((1, D), lambda i: (0, 0)),
        ],
        out_specs=pl.BlockSpec((BN_ROWS, D), lambda i: (i, 0)),
        out_shape=jax.ShapeDtypeStruct((N, D), jnp.float32),
    )


def kernel(x, edge_index, W_l1, b_l1, W_r1, gamma1, beta1,
           W_l2, b_l2, W_r2, gamma2, beta2,
           W_l3, b_l3, W_r3, gamma3, beta3):
    N, D0 = x.shape
    E = edge_index.shape[1]
    D1 = W_l1.shape[1]
    D2 = W_l2.shape[1]
    D3 = W_l3.shape[1]
    nchunk = E // CH

    ei = edge_index.astype(jnp.int32)
    need = max(_chunk_starts(nchunk, NS)[0], _chunk_starts(nchunk, NW)[0])
    src2d = ei[0].reshape(nchunk, CH)
    dst2d = ei[1].reshape(nchunk, CH)
    if need > nchunk:
        padcfg = ((0, need - nchunk), (0, 0))
        src2d = jnp.pad(src2d, padcfg)
        dst2d = jnp.pad(dst2d, padcfg)

    row = lambda v: v.reshape(1, -1)

    agg128 = _make_agg_kernel(N, E, D0)

    cnt1 = agg128(jnp.ones((N, D0), jnp.float32), src2d, dst2d)
    agg1 = agg128(x, src2d, dst2d)
    pre1, s1a, s1b, inv2d = _layer1_a(N, D0, D1)(
        agg1, cnt1, x, W_l1, W_r1, row(b_l1))
    p2a, p2b, r2 = _layer_b_next(N, D1, D2, split_p=True)(
        pre1, s1a, s1b, row(gamma1), row(beta1), W_l2, W_r2)

    agg2a = agg128(p2a, src2d, dst2d)
    agg2b = agg128(p2b, src2d, dst2d)
    pre2, s2a, s2b = _layer_a_nomm(N, D2, nparts=2)(
        agg2a, agg2b, inv2d, r2, row(b_l2))
    p3, r3 = _layer_b_next(N, D2, D3, split_p=False)(
        pre2, s2a, s2b, row(gamma2), row(beta2), W_l3, W_r3)

    agg3 = agg128(p3, src2d, dst2d)
    pre3, s3a, s3b = _layer_a_nomm(N, D3, nparts=1)(
        agg3, inv2d, r3, row(b_l3))
    return _layer_b_final(N, D3)(pre3, s3a, s3b, row(gamma3), row(beta3))
